# baseline (speedup 1.0000x reference)
import functools

import jax
import jax.numpy as jnp
from jax import lax
from jax.experimental import pallas as pl
from jax.experimental.pallas import tpu as pltpu
from jax.experimental.pallas import tpu_sc as plsc

K, R, KAPPA, NX, NF, DOUT = 4, 16384, 32, 3, 13, 16
NIN = NX + NF
NPTS = K * R
NIDX = NPTS * KAPPA

NC, NS = 2, 16
NW = NC * NS
PTS_PER_W = NPTS // NW

IDX_PER_DMA = 128
PTS_PER_DMA = IDX_PER_DMA // KAPPA
DMAS_PER_BUF = 16
PTS_PER_BUF = DMAS_PER_BUF * PTS_PER_DMA
BUFS_PER_W = PTS_PER_W // PTS_PER_BUF
IDX_ROWS_TOTAL = NIDX // IDX_PER_DMA
IDX_ROWS_PER_W = PTS_PER_W * KAPPA // IDX_PER_DMA

PPR = 128 // NIN
NROWS8 = NPTS // PPR
ROWS_BLK = 2048


def _mlp_body(p_ref, h_ref, z_ref):
    z_ref[...] = jnp.maximum(
        jnp.dot(p_ref[...], h_ref[...], preferred_element_type=jnp.float32),
        0.0).astype(jnp.bfloat16)


def _dense_body(a_ref, b_ref, g_ref, bias_ref, y_ref):
    m = jnp.maximum(a_ref[...], b_ref[...]).astype(jnp.float32)
    y_ref[...] = (
        jnp.dot(m, g_ref[...], preferred_element_type=jnp.float32)
        + bias_ref[...])


_sc_mesh = plsc.VectorSubcoreMesh(core_axis_name="c", subcore_axis_name="s")


HALF_BUFS = BUFS_PER_W // 2


@functools.partial(
    pl.kernel,
    out_type=jax.ShapeDtypeStruct((NPTS, 2, DOUT), jnp.bfloat16),
    mesh=_sc_mesh,
    compiler_params=pltpu.CompilerParams(use_tc_tiling_on_sc=False),
    scratch_types=[
        pltpu.VMEM((2, DMAS_PER_BUF, IDX_PER_DMA), jnp.int32),
        pltpu.VMEM((2, PTS_PER_BUF * KAPPA, DOUT), jnp.bfloat16),
        pltpu.VMEM((2, PTS_PER_BUF, 2, DOUT), jnp.bfloat16),
        pltpu.SemaphoreType.DMA,
        pltpu.SemaphoreType.DMA,
    ],
)
def _gather_max(z_hbm, nidx_hbm, out_hbm, idx_v, rows_v, out_v, sem0, sem1):
    wid = lax.axis_index("s") * NC + lax.axis_index("c")
    idx_row0 = wid * IDX_ROWS_PER_W
    pt0 = wid * PTS_PER_W
    kbase = (pt0 // R) * R
    z_batch = z_hbm.at[pl.ds(kbase, R)]
    sems = (sem0, sem1)

    def fire(parity, t):
        pltpu.sync_copy(
            nidx_hbm.at[pl.ds(idx_row0 + t * DMAS_PER_BUF, DMAS_PER_BUF)],
            idx_v.at[parity])
        for j in range(DMAS_PER_BUF):
            pltpu.async_copy(
                z_batch.at[idx_v.at[parity, j]],
                rows_v.at[parity, pl.ds(j * IDX_PER_DMA, IDX_PER_DMA)],
                sems[parity])

    def drain(parity):
        pltpu.make_async_copy(
            z_hbm.at[pl.ds(0, PTS_PER_BUF * KAPPA)],
            rows_v.at[parity], sems[parity]).wait()

    def compute(parity, t):
        def pt_body(p, c):
            base = p * KAPPA
            acc = rows_v[parity, pl.ds(base, 2), :]
            for i in range(1, KAPPA // 2):
                acc = jnp.maximum(acc, rows_v[parity, pl.ds(base + 2 * i, 2), :])
            out_v[parity, p] = acc
            return c

        lax.fori_loop(0, PTS_PER_BUF, pt_body, 0, unroll=2)
        pltpu.sync_copy(
            out_v.at[parity],
            out_hbm.at[pl.ds(pt0 + t * PTS_PER_BUF, PTS_PER_BUF)])

    fire(0, 0)
    fire(1, 1)

    def buf_body(t2, carry):
        drain(0)
        compute(0, 2 * t2)

        @pl.when(t2 < HALF_BUFS - 1)
        def _():
            fire(0, 2 * t2 + 2)

        drain(1)
        compute(1, 2 * t2 + 1)

        @pl.when(t2 < HALF_BUFS - 1)
        def _():
            fire(1, 2 * t2 + 3)

        return carry

    lax.fori_loop(0, HALF_BUFS, buf_body, 0)


def kernel(X, F, N, H, Gamma, gamma_bias):
    pflat = jnp.concatenate([X, F], axis=2).reshape(NROWS8, PPR * NIN)
    nflat = N.reshape(IDX_ROWS_TOTAL, IDX_PER_DMA)
    eye8 = jnp.eye(PPR, dtype=jnp.float32)
    h_bd = jnp.kron(eye8, H.T)
    g_bd = jnp.kron(eye8, Gamma)
    b_t = jnp.tile(gamma_bias, PPR)[None, :]

    z = pl.pallas_call(
        _mlp_body,
        grid=(NROWS8 // ROWS_BLK,),
        in_specs=[
            pl.BlockSpec((ROWS_BLK, 128), lambda i: (i, 0)),
            pl.BlockSpec((128, 128), lambda i: (0, 0)),
        ],
        out_specs=pl.BlockSpec((ROWS_BLK, 128), lambda i: (i, 0)),
        out_shape=jax.ShapeDtypeStruct((NROWS8, 128), jnp.bfloat16),
    )(pflat, h_bd)

    y0 = _gather_max(z.reshape(NPTS, DOUT), nflat)

    y0a = y0[:, 0, :].reshape(NROWS8, PPR * DOUT)
    y0b = y0[:, 1, :].reshape(NROWS8, PPR * DOUT)
    y = pl.pallas_call(
        _dense_body,
        grid=(NROWS8 // ROWS_BLK,),
        in_specs=[
            pl.BlockSpec((ROWS_BLK, 128), lambda i: (i, 0)),
            pl.BlockSpec((ROWS_BLK, 128), lambda i: (i, 0)),
            pl.BlockSpec((128, 128), lambda i: (0, 0)),
            pl.BlockSpec((1, 128), lambda i: (0, 0)),
        ],
        out_specs=pl.BlockSpec((ROWS_BLK, 128), lambda i: (i, 0)),
        out_shape=jax.ShapeDtypeStruct((NROWS8, 128), jnp.float32),
    )(y0a, y0b, g_bd, b_t)

    return y.reshape(K, R, DOUT)


# speedup vs baseline: 4.4267x; 4.4267x over previous
import functools

import jax
import jax.numpy as jnp
from jax import lax
from jax.experimental import pallas as pl
from jax.experimental.pallas import tpu as pltpu
from jax.experimental.pallas import tpu_sc as plsc

K, R, KAPPA, NX, NF, DOUT = 4, 16384, 32, 3, 13, 16
NIN = NX + NF
NPTS = K * R
NIDX = NPTS * KAPPA

NC, NS = 2, 16
NW = NC * NS
PTS_PER_W = NPTS // NW

IDX_PER_DMA = 128
PTS_PER_DMA = IDX_PER_DMA // KAPPA
DMAS_PER_BUF = 16
PTS_PER_BUF = DMAS_PER_BUF * PTS_PER_DMA
BUFS_PER_W = PTS_PER_W // PTS_PER_BUF
IDX_ROWS_TOTAL = NIDX // IDX_PER_DMA
IDX_ROWS_PER_W = PTS_PER_W * KAPPA // IDX_PER_DMA

PPR = 128 // NIN
NROWS8 = NPTS // PPR
ROWS_BLK = 2048


def _mlp_body(p_ref, h_ref, z_ref):
    z_ref[...] = jnp.maximum(
        jnp.dot(p_ref[...], h_ref[...], preferred_element_type=jnp.float32), 0.0)


def _dense_body(p_ref, g_ref, b_ref, y_ref):
    y_ref[...] = (
        jnp.dot(p_ref[...], g_ref[...], preferred_element_type=jnp.float32)
        + b_ref[...])


_sc_mesh = plsc.VectorSubcoreMesh(core_axis_name="c", subcore_axis_name="s")

HALF_BUFS = BUFS_PER_W // 2


@functools.partial(
    pl.kernel,
    out_type=jax.ShapeDtypeStruct((NPTS, DOUT), jnp.float32),
    mesh=_sc_mesh,
    compiler_params=pltpu.CompilerParams(use_tc_tiling_on_sc=False),
    scratch_types=[
        pltpu.VMEM((2, DMAS_PER_BUF, IDX_PER_DMA), jnp.int32),
        pltpu.VMEM((2, PTS_PER_BUF * KAPPA, DOUT), jnp.float32),
        pltpu.VMEM((2, PTS_PER_BUF, DOUT), jnp.float32),
        pltpu.SemaphoreType.DMA,
        pltpu.SemaphoreType.DMA,
    ],
)
def _gather_max(z_hbm, nidx_hbm, out_hbm, idx_v, rows_v, out_v, sem0, sem1):
    wid = lax.axis_index("s") * NC + lax.axis_index("c")
    idx_row0 = wid * IDX_ROWS_PER_W
    pt0 = wid * PTS_PER_W
    kbase = (pt0 // R) * R
    z_batch = z_hbm.at[pl.ds(kbase, R)]
    sems = (sem0, sem1)

    def fire(parity, t):
        pltpu.sync_copy(
            nidx_hbm.at[pl.ds(idx_row0 + t * DMAS_PER_BUF, DMAS_PER_BUF)],
            idx_v.at[parity])
        for j in range(DMAS_PER_BUF):
            pltpu.async_copy(
                z_batch.at[idx_v.at[parity, j]],
                rows_v.at[parity, pl.ds(j * IDX_PER_DMA, IDX_PER_DMA)],
                sems[parity])

    def drain(parity):
        pltpu.make_async_copy(
            z_hbm.at[pl.ds(0, PTS_PER_BUF * KAPPA)],
            rows_v.at[parity], sems[parity]).wait()

    def compute(parity, t):
        def pt_body(p, c):
            base = p * KAPPA
            acc = rows_v[parity, base]
            for i in range(1, KAPPA):
                acc = jnp.maximum(acc, rows_v[parity, base + i])
            out_v[parity, p] = acc
            return c

        lax.fori_loop(0, PTS_PER_BUF, pt_body, 0, unroll=4)
        pltpu.sync_copy(
            out_v.at[parity],
            out_hbm.at[pl.ds(pt0 + t * PTS_PER_BUF, PTS_PER_BUF)])

    fire(0, 0)
    fire(1, 1)

    def buf_body(t2, carry):
        drain(0)
        compute(0, 2 * t2)

        @pl.when(t2 < HALF_BUFS - 1)
        def _():
            fire(0, 2 * t2 + 2)

        drain(1)
        compute(1, 2 * t2 + 1)

        @pl.when(t2 < HALF_BUFS - 1)
        def _():
            fire(1, 2 * t2 + 3)

        return carry

    lax.fori_loop(0, HALF_BUFS, buf_body, 0)


def kernel(X, F, N, H, Gamma, gamma_bias):
    pflat = jnp.concatenate([X, F], axis=2).reshape(NROWS8, PPR * NIN)
    nflat = N.reshape(IDX_ROWS_TOTAL, IDX_PER_DMA)
    eye8 = jnp.eye(PPR, dtype=jnp.float32)
    h_bd = jnp.kron(eye8, H.T)
    g_bd = jnp.kron(eye8, Gamma)
    b_t = jnp.tile(gamma_bias, PPR)[None, :]

    z = pl.pallas_call(
        _mlp_body,
        grid=(NROWS8 // ROWS_BLK,),
        in_specs=[
            pl.BlockSpec((ROWS_BLK, 128), lambda i: (i, 0)),
            pl.BlockSpec((128, 128), lambda i: (0, 0)),
        ],
        out_specs=pl.BlockSpec((ROWS_BLK, 128), lambda i: (i, 0)),
        out_shape=jax.ShapeDtypeStruct((NROWS8, 128), jnp.float32),
    )(pflat, h_bd)

    y0 = _gather_max(z.reshape(NPTS, DOUT), nflat)

    y = pl.pallas_call(
        _dense_body,
        grid=(NROWS8 // ROWS_BLK,),
        in_specs=[
            pl.BlockSpec((ROWS_BLK, 128), lambda i: (i, 0)),
            pl.BlockSpec((128, 128), lambda i: (0, 0)),
            pl.BlockSpec((1, 128), lambda i: (0, 0)),
        ],
        out_specs=pl.BlockSpec((ROWS_BLK, 128), lambda i: (i, 0)),
        out_shape=jax.ShapeDtypeStruct((NROWS8, 128), jnp.float32),
    )(y0.reshape(NROWS8, PPR * DOUT), g_bd, b_t)

    return y.reshape(K, R, DOUT)


# speedup vs baseline: 4.4393x; 1.0029x over previous
import functools

import jax
import jax.numpy as jnp
from jax import lax
from jax.experimental import pallas as pl
from jax.experimental.pallas import tpu as pltpu
from jax.experimental.pallas import tpu_sc as plsc

K, R, KAPPA, NX, NF, DOUT = 4, 16384, 32, 3, 13, 16
NIN = NX + NF
NPTS = K * R
NIDX = NPTS * KAPPA

NC, NS = 2, 16
NW = NC * NS
PTS_PER_W = NPTS // NW

IDX_PER_DMA = 128
PTS_PER_DMA = IDX_PER_DMA // KAPPA
DMAS_PER_BUF = 16
PTS_PER_BUF = DMAS_PER_BUF * PTS_PER_DMA
BUFS_PER_W = PTS_PER_W // PTS_PER_BUF
IDX_ROWS_TOTAL = NIDX // IDX_PER_DMA
IDX_ROWS_PER_W = PTS_PER_W * KAPPA // IDX_PER_DMA

PPR = 128 // NIN
NROWS8 = NPTS // PPR
ROWS_BLK = 2048


def _mlp_body(p_ref, h_ref, z_ref):
    z_ref[...] = jnp.maximum(
        jnp.dot(p_ref[...], h_ref[...], preferred_element_type=jnp.float32), 0.0)


def _dense_body(p_ref, g_ref, b_ref, y_ref):
    y_ref[...] = (
        jnp.dot(p_ref[...], g_ref[...], preferred_element_type=jnp.float32)
        + b_ref[...])


_sc_mesh = plsc.VectorSubcoreMesh(core_axis_name="c", subcore_axis_name="s")

HALF_BUFS = BUFS_PER_W // 2


@functools.partial(
    pl.kernel,
    out_type=jax.ShapeDtypeStruct((NPTS, DOUT), jnp.float32),
    mesh=_sc_mesh,
    compiler_params=pltpu.CompilerParams(use_tc_tiling_on_sc=False),
    scratch_types=[
        pltpu.VMEM((2, DMAS_PER_BUF, IDX_PER_DMA), jnp.int32),
        pltpu.VMEM((2, PTS_PER_BUF * KAPPA, DOUT), jnp.float32),
        pltpu.VMEM((2, PTS_PER_BUF, DOUT), jnp.float32),
        pltpu.SemaphoreType.DMA,
        pltpu.SemaphoreType.DMA,
    ],
)
def _gather_max(z_hbm, nidx_hbm, out_hbm, idx_v, rows_v, out_v, sem0, sem1):
    wid = lax.axis_index("s") * NC + lax.axis_index("c")
    idx_row0 = wid * IDX_ROWS_PER_W
    pt0 = wid * PTS_PER_W
    kbase = (pt0 // R) * R
    z_batch = z_hbm.at[pl.ds(kbase, R)]
    sems = (sem0, sem1)

    def fire(parity, t):
        pltpu.sync_copy(
            nidx_hbm.at[pl.ds(idx_row0 + t * DMAS_PER_BUF, DMAS_PER_BUF)],
            idx_v.at[parity])
        for j in range(DMAS_PER_BUF):
            pltpu.async_copy(
                z_batch.at[idx_v.at[parity, j]],
                rows_v.at[parity, pl.ds(j * IDX_PER_DMA, IDX_PER_DMA)],
                sems[parity])

    def drain(parity):
        pltpu.make_async_copy(
            z_hbm.at[pl.ds(0, PTS_PER_BUF * KAPPA)],
            rows_v.at[parity], sems[parity]).wait()

    def compute(parity, t):
        def pt_body(p, c):
            base = p * KAPPA
            acc = rows_v[parity, base]
            for i in range(1, KAPPA):
                acc = jnp.maximum(acc, rows_v[parity, base + i])
            out_v[parity, p] = acc
            return c

        lax.fori_loop(0, PTS_PER_BUF, pt_body, 0, unroll=2)
        pltpu.sync_copy(
            out_v.at[parity],
            out_hbm.at[pl.ds(pt0 + t * PTS_PER_BUF, PTS_PER_BUF)])

    fire(0, 0)
    fire(1, 1)

    def buf_body(t2, carry):
        drain(0)
        compute(0, 2 * t2)

        @pl.when(t2 < HALF_BUFS - 1)
        def _():
            fire(0, 2 * t2 + 2)

        drain(1)
        compute(1, 2 * t2 + 1)

        @pl.when(t2 < HALF_BUFS - 1)
        def _():
            fire(1, 2 * t2 + 3)

        return carry

    lax.fori_loop(0, HALF_BUFS, buf_body, 0)


def kernel(X, F, N, H, Gamma, gamma_bias):
    pflat = jnp.concatenate([X, F], axis=2).reshape(NROWS8, PPR * NIN)
    nflat = N.reshape(IDX_ROWS_TOTAL, IDX_PER_DMA)
    eye8 = jnp.eye(PPR, dtype=jnp.float32)
    h_bd = jnp.kron(eye8, H.T)
    g_bd = jnp.kron(eye8, Gamma)
    b_t = jnp.tile(gamma_bias, PPR)[None, :]

    z = pl.pallas_call(
        _mlp_body,
        grid=(NROWS8 // ROWS_BLK,),
        in_specs=[
            pl.BlockSpec((ROWS_BLK, 128), lambda i: (i, 0)),
            pl.BlockSpec((128, 128), lambda i: (0, 0)),
        ],
        out_specs=pl.BlockSpec((ROWS_BLK, 128), lambda i: (i, 0)),
        out_shape=jax.ShapeDtypeStruct((NROWS8, 128), jnp.float32),
    )(pflat, h_bd)

    y0 = _gather_max(z.reshape(NPTS, DOUT), nflat)

    y = pl.pallas_call(
        _dense_body,
        grid=(NROWS8 // ROWS_BLK,),
        in_specs=[
            pl.BlockSpec((ROWS_BLK, 128), lambda i: (i, 0)),
            pl.BlockSpec((128, 128), lambda i: (0, 0)),
            pl.BlockSpec((1, 128), lambda i: (0, 0)),
        ],
        out_specs=pl.BlockSpec((ROWS_BLK, 128), lambda i: (i, 0)),
        out_shape=jax.ShapeDtypeStruct((NROWS8, 128), jnp.float32),
    )(y0.reshape(NROWS8, PPR * DOUT), g_bd, b_t)

    return y.reshape(K, R, DOUT)
